# Initial kernel scaffold; baseline (speedup 1.0000x reference)
import jax
import jax.numpy as jnp
from jax.experimental import pallas as pl


def kernel(x, edge_index, edge_attr, batch, W_node, b_node, W_edge, b_edge, Wnn1, bnn1, Wr1, br1, Wnn2, bnn2, Wr2, br2, Wnn3, bnn3, Wr3, br3, W1, b1, W2, b2, W3, b3):
    raise NotImplementedError("write your pallas kernel here")


# speedup vs baseline: 1.1446x; 1.1446x over previous
import functools

import jax
import jax.numpy as jnp
from jax import lax
from jax.experimental import pallas as pl
from jax.experimental.pallas import tpu as pltpu
from jax.experimental.pallas import tpu_sc as plsc

H = 16
G = 64
NC = 2
NS = 16
NW = NC * NS
CH = 128
GS = 8

TE = 2000
TN = 1000


def _lin_relu_body(x_ref, w_ref, b_ref, o_ref):
    o_ref[...] = jax.nn.relu(
        jnp.dot(x_ref[...], w_ref[...], preferred_element_type=jnp.float32)
        + b_ref[...])


def _lin_relu(x, w, b, tile):
    n, k = x.shape
    m = w.shape[1]
    return pl.pallas_call(
        _lin_relu_body,
        grid=(n // tile,),
        in_specs=[pl.BlockSpec((tile, k), lambda i: (i, 0)),
                  pl.BlockSpec((k, m), lambda i: (0, 0)),
                  pl.BlockSpec((1, m), lambda i: (0, 0))],
        out_specs=pl.BlockSpec((tile, m), lambda i: (i, 0)),
        out_shape=jax.ShapeDtypeStruct((n, m), jnp.float32),
    )(x, w, b.reshape(1, m))


def _msg_body(ea_ref, xj_ref, wnn_ref, bnn_ref, o_ref):
    q = jnp.dot(ea_ref[...], wnn_ref[...],
                preferred_element_type=jnp.float32) + bnn_ref[...]
    xj = xj_ref[...]
    acc = xj[:, 0:1] * q[:, 0:H]
    for h in range(1, H):
        acc = acc + xj[:, h:h + 1] * q[:, h * H:(h + 1) * H]
    o_ref[...] = acc


def _msg(ea, xj, wnn, bnn, e, ep):
    return pl.pallas_call(
        _msg_body,
        grid=(e // TE,),
        in_specs=[pl.BlockSpec((TE, H), lambda i: (i, 0)),
                  pl.BlockSpec((TE, H), lambda i: (i, 0)),
                  pl.BlockSpec((H, H * H), lambda i: (0, 0)),
                  pl.BlockSpec((1, H * H), lambda i: (0, 0))],
        out_specs=pl.BlockSpec((TE, H), lambda i: (i, 0)),
        out_shape=jax.ShapeDtypeStruct((ep, H), jnp.float32),
    )(ea, xj, wnn, bnn.reshape(1, H * H))


def _update_body(parts_ref, xin_ref, wr_ref, br_ref, o_ref):
    xin = xin_ref[...]
    lin = jnp.dot(xin, wr_ref[...],
                  preferred_element_type=jnp.float32) + br_ref[...]
    agg = parts_ref[0] + parts_ref[1]
    o_ref[...] = jax.nn.relu(agg + lin) + xin


def _update(parts, xin, wr, br, n, nsp):
    return pl.pallas_call(
        _update_body,
        grid=(n // TN,),
        in_specs=[pl.BlockSpec((NC, TN, H), lambda i: (0, i, 0)),
                  pl.BlockSpec((TN, H), lambda i: (i, 0)),
                  pl.BlockSpec((H, H), lambda i: (0, 0)),
                  pl.BlockSpec((1, H), lambda i: (0, 0))],
        out_specs=pl.BlockSpec((TN, H), lambda i: (i, 0)),
        out_shape=jax.ShapeDtypeStruct((n, H), jnp.float32),
    )(parts, xin, wr, br.reshape(1, H))


def _pool_mlp_body(x_ref, b3_ref, w1_ref, b1_ref, w2_ref, b2_ref,
                   w3_ref, b3b_ref, o_ref, acc_ref):
    i = pl.program_id(0)

    @pl.when(i == 0)
    def _():
        acc_ref[...] = jnp.zeros_like(acc_ref)

    bt = b3_ref[0]
    gids = lax.broadcasted_iota(jnp.int32, (G, TN), 0)
    mask = (bt == gids).astype(jnp.float32)
    acc_ref[...] += jnp.dot(mask, x_ref[...],
                            preferred_element_type=jnp.float32)

    @pl.when(i == pl.num_programs(0) - 1)
    def _():
        p = acc_ref[...]
        o1 = jax.nn.relu(jnp.dot(p, w1_ref[...],
                                 preferred_element_type=jnp.float32)
                         + b1_ref[...])
        o2 = jax.nn.relu(jnp.dot(o1, w2_ref[...],
                                 preferred_element_type=jnp.float32)
                         + b2_ref[...])
        o_ref[...] = jnp.dot(o2, w3_ref[...],
                             preferred_element_type=jnp.float32) + b3b_ref[...]


def _pool_mlp(x3s, batch, w1, b1, w2, b2, w3, b3, n):
    nb = n // TN
    batch3 = batch.reshape(nb, 1, TN)
    d1, d2, d3 = w1.shape[1], w2.shape[1], w3.shape[1]
    return pl.pallas_call(
        _pool_mlp_body,
        grid=(nb,),
        in_specs=[pl.BlockSpec((TN, H), lambda i: (i, 0)),
                  pl.BlockSpec((1, 1, TN), lambda i: (i, 0, 0)),
                  pl.BlockSpec((H, d1), lambda i: (0, 0)),
                  pl.BlockSpec((1, d1), lambda i: (0, 0)),
                  pl.BlockSpec((d1, d2), lambda i: (0, 0)),
                  pl.BlockSpec((1, d2), lambda i: (0, 0)),
                  pl.BlockSpec((d2, d3), lambda i: (0, 0)),
                  pl.BlockSpec((1, d3), lambda i: (0, 0))],
        out_specs=pl.BlockSpec((G, d3), lambda i: (0, 0)),
        out_shape=jax.ShapeDtypeStruct((G, d3), jnp.float32),
        scratch_shapes=[pltpu.VMEM((G, H), jnp.float32)],
    )(x3s, batch3, w1, b1.reshape(1, d1), w2, b2.reshape(1, d2),
      w3, b3.reshape(1, d3))


def _sc_gather(xin, idx3, n, ep):
    epw = ep // NW
    nch = epw // CH
    mesh = plsc.VectorSubcoreMesh(core_axis_name="c", subcore_axis_name="s")

    @functools.partial(
        pl.kernel,
        out_type=jax.ShapeDtypeStruct((ep, H), jnp.float32),
        mesh=mesh,
        scratch_types=[pltpu.VMEM((nch, CH), jnp.int32),
                       pltpu.VMEM((epw, H), jnp.float32),
                       pltpu.SemaphoreType.DMA],
        compiler_params=pltpu.CompilerParams(use_tc_tiling_on_sc=False),
    )
    def k(x_hbm, idx_hbm, out_hbm, idx_v, rows_v, sem):
        wid = lax.axis_index("s") * NC + lax.axis_index("c")
        pltpu.sync_copy(idx_hbm.at[wid], idx_v)

        def grp(g, carry):
            descs = []
            for jj in range(GS):
                j = g * GS + jj
                descs.append(pltpu.async_copy(
                    x_hbm.at[idx_v.at[j]],
                    rows_v.at[pl.ds(j * CH, CH)], sem))
            for d in descs:
                d.wait()
            return carry

        lax.fori_loop(0, nch // GS, grp, 0)
        pltpu.sync_copy(rows_v, out_hbm.at[pl.ds(wid * epw, epw)])

    return k(xin, idx3)


def _sc_scatter(msg, dst3, zeros_hbm, nsp, ep):
    epw = ep // NW
    nch = epw // CH
    rps = nsp // NS
    mesh = plsc.VectorSubcoreMesh(core_axis_name="c", subcore_axis_name="s")

    @functools.partial(
        pl.kernel,
        out_type=jax.ShapeDtypeStruct((NC, nsp, H), jnp.float32),
        mesh=mesh,
        scratch_types=[pltpu.VMEM((nch, CH), jnp.int32),
                       pltpu.VMEM((epw, H), jnp.float32),
                       pltpu.VMEM_SHARED((nsp, H), jnp.float32),
                       pltpu.SemaphoreType.DMA],
        compiler_params=pltpu.CompilerParams(use_tc_tiling_on_sc=False),
    )
    def k(msg_hbm, dst_hbm, z_hbm, out_hbm, dst_v, msg_v, agg_sh, sem):
        c = lax.axis_index("c")
        s = lax.axis_index("s")
        wid = s * NC + c
        pltpu.sync_copy(dst_hbm.at[wid], dst_v)
        pltpu.sync_copy(msg_hbm.at[pl.ds(wid * epw, epw)], msg_v)
        pltpu.sync_copy(z_hbm, agg_sh.at[pl.ds(s * rps, rps)])
        plsc.subcore_barrier()

        def grp(g, carry):
            descs = []
            for jj in range(GS):
                j = g * GS + jj
                descs.append(pltpu.async_copy(
                    msg_v.at[pl.ds(j * CH, CH)],
                    agg_sh.at[dst_v.at[j]], sem, add=True))
            for d in descs:
                d.wait()
            return carry

        lax.fori_loop(0, nch // GS, grp, 0)
        plsc.subcore_barrier()
        pltpu.sync_copy(agg_sh.at[pl.ds(s * rps, rps)],
                        out_hbm.at[c].at[pl.ds(s * rps, rps)])

    return k(msg, dst3, zeros_hbm)


def kernel(x, edge_index, edge_attr, batch,
           W_node, b_node, W_edge, b_edge,
           Wnn1, bnn1, Wr1, br1,
           Wnn2, bnn2, Wr2, br2,
           Wnn3, bnn3, Wr3, br3,
           W1, b1, W2, b2, W3, b3):
    n = x.shape[0]
    e = edge_index.shape[1]
    ep = ((e + NW * CH - 1) // (NW * CH)) * (NW * CH)
    nsp = ((n // NS) + CH) // CH * CH * NS
    epw = ep // NW
    nch = epw // CH
    rps = nsp // NS

    src = edge_index[0]
    dst = edge_index[1]
    pad = ep - e
    src3 = jnp.concatenate(
        [src, jnp.zeros((pad,), jnp.int32)]).reshape(NW, nch, CH)
    dst3 = jnp.concatenate(
        [dst, jnp.full((pad,), n, jnp.int32)]).reshape(NW, nch, CH)
    zeros_hbm = jnp.zeros((rps, H), jnp.float32)

    h = _lin_relu(x, W_node, b_node, TN)
    ea = _lin_relu(edge_attr, W_edge, b_edge, TE)

    xin = h
    for wnn, bnn, wr, br in ((Wnn1, bnn1, Wr1, br1),
                             (Wnn2, bnn2, Wr2, br2),
                             (Wnn3, bnn3, Wr3, br3)):
        xj = _sc_gather(xin, src3, n, ep)
        msg = _msg(ea, xj, wnn, bnn, e, ep)
        parts = _sc_scatter(msg, dst3, zeros_hbm, nsp, ep)
        xin = _update(parts, xin, wr, br, n, nsp)

    return _pool_mlp(xin, batch, W1, b1, W2, b2, W3, b3, n)
